# Initial kernel scaffold; baseline (speedup 1.0000x reference)
import functools

import jax
import jax.numpy as jnp
from jax import lax
from jax.experimental import pallas as pl
from jax.experimental.pallas import tpu as pltpu
from jax.experimental.pallas import tpu_sc as plsc

N = 10000
E = 320000
D_IN = 128
D_H = 128
D_OUT = 40
DP = 48
NP = 10240

NC = 2
NS = 16
NW = NC * NS
EPW = E // NW
CH = 80
NCH = EPW // CH
RPT = NP // NS
ZCH = 80
NZ = RPT // ZCH

WD = 16

BR = 1024
GR = NP // BR


def _sc_mesh():
    return plsc.VectorSubcoreMesh(core_axis_name="c", subcore_axis_name="s")


def _make_agg(D):
    @functools.partial(
        pl.kernel,
        out_type=jax.ShapeDtypeStruct((NC * NP, D), jnp.float32),
        mesh=_sc_mesh(),
        scratch_types=[
            pltpu.VMEM_SHARED((NP, D), jnp.float32),
            pltpu.VMEM((CH,), jnp.int32),
            pltpu.VMEM((CH,), jnp.int32),
            pltpu.VMEM((CH, D), jnp.float32),
            pltpu.VMEM((ZCH, D), jnp.float32),
            pltpu.SemaphoreType.DMA,
        ],
    )
    def agg(s_hbm, src_hbm, dst_hbm, zeros_hbm, out_hbm,
            shared, idx_v, didx_v, rows_v, zv, sem):
        cid = lax.axis_index("c")
        sid = lax.axis_index("s")
        wid = sid * NC + cid

        pltpu.sync_copy(zeros_hbm, zv)
        for k in range(NZ):
            pltpu.sync_copy(zv, shared.at[pl.ds(sid * RPT + k * ZCH, ZCH)])
        plsc.subcore_barrier()

        base = wid * EPW

        def step(j, carry):
            off = pl.multiple_of(base + j * CH, 8)
            pltpu.sync_copy(src_hbm.at[pl.ds(off, CH)], idx_v)
            pltpu.sync_copy(dst_hbm.at[pl.ds(off, CH)], didx_v)
            pltpu.async_copy(s_hbm.at[idx_v], rows_v, sem).wait()
            pltpu.sync_copy(rows_v, shared.at[didx_v], add=True)
            return carry

        lax.fori_loop(0, NCH, step, 0)
        plsc.subcore_barrier()

        for k in range(NZ):
            row = sid * RPT + k * ZCH
            pltpu.sync_copy(shared.at[pl.ds(row, ZCH)], zv)
            pltpu.sync_copy(zv, out_hbm.at[pl.ds(cid * NP + row, ZCH)])

    return agg


@functools.partial(
    pl.kernel,
    out_type=jax.ShapeDtypeStruct((NC * NP, WD), jnp.float32),
    mesh=_sc_mesh(),
    scratch_types=[
        pltpu.VMEM_SHARED((NP, WD), jnp.float32),
        pltpu.VMEM((CH,), jnp.int32),
        pltpu.VMEM((CH, WD), jnp.float32),
        pltpu.VMEM((ZCH, WD), jnp.float32),
    ],
)
def _deg_kernel(dst_hbm, ones_hbm, zeros_hbm, out_hbm,
                shared, didx_v, ones_v, zv):
    cid = lax.axis_index("c")
    sid = lax.axis_index("s")
    wid = sid * NC + cid

    pltpu.sync_copy(zeros_hbm, zv)
    pltpu.sync_copy(ones_hbm, ones_v)
    for k in range(NZ):
        pltpu.sync_copy(zv, shared.at[pl.ds(sid * RPT + k * ZCH, ZCH)])
    plsc.subcore_barrier()

    base = wid * EPW

    def step(j, carry):
        off = pl.multiple_of(base + j * CH, 8)
        pltpu.sync_copy(dst_hbm.at[pl.ds(off, CH)], didx_v)
        pltpu.sync_copy(ones_v, shared.at[didx_v], add=True)
        return carry

    lax.fori_loop(0, NCH, step, 0)
    plsc.subcore_barrier()

    for k in range(NZ):
        row = sid * RPT + k * ZCH
        pltpu.sync_copy(shared.at[pl.ds(row, ZCH)], zv)
        pltpu.sync_copy(zv, out_hbm.at[pl.ds(cid * NP + row, ZCH)])


def _mm1_body(x_ref, w_ref, d0_ref, d1_ref, s_ref, dinv_ref):
    deg = d0_ref[:, :1] + d1_ref[:, :1]
    dinv = lax.rsqrt(deg + 1.0)
    h = jnp.dot(x_ref[...], w_ref[...], preferred_element_type=jnp.float32)
    s_ref[...] = h * dinv
    dinv_ref[...] = jnp.broadcast_to(dinv, dinv_ref.shape)


def _mm1(xp, W1, deg0, deg1):
    return pl.pallas_call(
        _mm1_body,
        grid=(GR,),
        in_specs=[
            pl.BlockSpec((BR, D_IN), lambda i: (i, 0)),
            pl.BlockSpec((D_IN, D_H), lambda i: (0, 0)),
            pl.BlockSpec((BR, WD), lambda i: (i, 0)),
            pl.BlockSpec((BR, WD), lambda i: (i, 0)),
        ],
        out_specs=[
            pl.BlockSpec((BR, D_H), lambda i: (i, 0)),
            pl.BlockSpec((BR, 8), lambda i: (i, 0)),
        ],
        out_shape=[
            jax.ShapeDtypeStruct((NP, D_H), jnp.float32),
            jax.ShapeDtypeStruct((NP, 8), jnp.float32),
        ],
    )(xp, W1, deg0, deg1)


def _mid_body(a0_ref, a1_ref, s_ref, dinv_ref, b_ref, w_ref, o_ref):
    dinv = dinv_ref[:, :1]
    t = (a0_ref[...] + a1_ref[...] + s_ref[...]) * dinv + b_ref[...]
    t = jnp.maximum(t, 0.0)
    o_ref[...] = jnp.dot(t, w_ref[...],
                         preferred_element_type=jnp.float32) * dinv


def _mm_mid(a0, a1, s, dinv, b, W):
    d_in = s.shape[1]
    d_out = W.shape[1]
    return pl.pallas_call(
        _mid_body,
        grid=(GR,),
        in_specs=[
            pl.BlockSpec((BR, d_in), lambda i: (i, 0)),
            pl.BlockSpec((BR, d_in), lambda i: (i, 0)),
            pl.BlockSpec((BR, d_in), lambda i: (i, 0)),
            pl.BlockSpec((BR, 8), lambda i: (i, 0)),
            pl.BlockSpec((1, d_in), lambda i: (0, 0)),
            pl.BlockSpec((d_in, d_out), lambda i: (0, 0)),
        ],
        out_specs=pl.BlockSpec((BR, d_out), lambda i: (i, 0)),
        out_shape=jax.ShapeDtypeStruct((NP, d_out), jnp.float32),
    )(a0, a1, s, dinv, b, W)


def _fin_body(a0_ref, a1_ref, s_ref, dinv_ref, b_ref, o_ref):
    dinv = dinv_ref[:, :1]
    t = (a0_ref[...] + a1_ref[...] + s_ref[...]) * dinv + b_ref[...]
    col = lax.broadcasted_iota(jnp.int32, t.shape, 1)
    valid = col < D_OUT
    tm = jnp.where(valid, t, -jnp.inf)
    m = jnp.max(tm, axis=1, keepdims=True)
    ex = jnp.where(valid, jnp.exp(t - m), 0.0)
    lse = jnp.log(jnp.sum(ex, axis=1, keepdims=True))
    o_ref[...] = t - m - lse


def _fin(a0, a1, s, dinv, b):
    return pl.pallas_call(
        _fin_body,
        grid=(GR,),
        in_specs=[
            pl.BlockSpec((BR, DP), lambda i: (i, 0)),
            pl.BlockSpec((BR, DP), lambda i: (i, 0)),
            pl.BlockSpec((BR, DP), lambda i: (i, 0)),
            pl.BlockSpec((BR, 8), lambda i: (i, 0)),
            pl.BlockSpec((1, DP), lambda i: (0, 0)),
        ],
        out_specs=pl.BlockSpec((BR, DP), lambda i: (i, 0)),
        out_shape=jax.ShapeDtypeStruct((NP, DP), jnp.float32),
    )(a0, a1, s, dinv, b)


_agg_h = _make_agg(D_H)
_agg_p = _make_agg(DP)


@jax.jit
def kernel(x, edge_index, W1, b1, W2, b2, Wo, bo):
    src = edge_index[0].astype(jnp.int32)
    dst = edge_index[1].astype(jnp.int32)
    xp = jnp.pad(x, ((0, NP - N), (0, 0)))
    Wop = jnp.pad(Wo, ((0, 0), (0, DP - D_OUT)))
    b1r = b1.reshape(1, D_H)
    b2r = b2.reshape(1, D_H)
    bor = jnp.pad(bo, (0, DP - D_OUT)).reshape(1, DP)

    ones_w = jnp.ones((CH, WD), jnp.float32)
    zeros_w = jnp.zeros((ZCH, WD), jnp.float32)
    zeros_h = jnp.zeros((ZCH, D_H), jnp.float32)
    zeros_p = jnp.zeros((ZCH, DP), jnp.float32)

    degp = _deg_kernel(dst, ones_w, zeros_w)
    deg0, deg1 = degp[:NP], degp[NP:]

    s1, dinv = _mm1(xp, W1, deg0, deg1)
    a1 = _agg_h(s1, src, dst, zeros_h)
    s2 = _mm_mid(a1[:NP], a1[NP:], s1, dinv, b1r, W2)
    a2 = _agg_h(s2, src, dst, zeros_h)
    s3 = _mm_mid(a2[:NP], a2[NP:], s2, dinv, b2r, Wop)
    a3 = _agg_p(s3, src, dst, zeros_p)
    outp = _fin(a3[:NP], a3[NP:], s3, dinv, bor)
    return outp[:N, :D_OUT]


# speedup vs baseline: 10.2831x; 10.2831x over previous
import functools

import jax
import jax.numpy as jnp
from jax import lax
from jax.experimental import pallas as pl
from jax.experimental.pallas import tpu as pltpu
from jax.experimental.pallas import tpu_sc as plsc

N = 10000
E = 320000
D_IN = 128
D_H = 128
D_OUT = 40
DP = 128
NP = 10240

NC = 2
NS = 16
NW = NC * NS
EPW = E // NW
CH = 80
NCH = EPW // CH
RPT = NP // NS
ZCH = 80
NZ = RPT // ZCH

WD = 128

BR = 1024
GR = NP // BR


def _sc_mesh():
    return plsc.VectorSubcoreMesh(core_axis_name="c", subcore_axis_name="s")


def _make_agg(D):
    @functools.partial(
        pl.kernel,
        out_type=jax.ShapeDtypeStruct((NC * NP, D), jnp.float32),
        mesh=_sc_mesh(),
        scratch_types=[
            pltpu.VMEM_SHARED((NP, D), jnp.float32),
            pltpu.VMEM((CH,), jnp.int32),
            pltpu.VMEM((CH,), jnp.int32),
            pltpu.VMEM((CH, D), jnp.float32),
            pltpu.VMEM((ZCH, D), jnp.float32),
            pltpu.SemaphoreType.DMA,
        ],
    )
    def agg(s_hbm, src_hbm, dst_hbm, zeros_hbm, out_hbm,
            shared, idx_v, didx_v, rows_v, zv, sem):
        cid = lax.axis_index("c")
        sid = lax.axis_index("s")
        wid = sid * NC + cid

        pltpu.sync_copy(zeros_hbm, zv)
        for k in range(NZ):
            pltpu.sync_copy(zv, shared.at[pl.ds(sid * RPT + k * ZCH, ZCH)])
        plsc.subcore_barrier()

        base = wid * EPW

        def step(j, carry):
            off = pl.multiple_of(base + j * CH, 8)
            pltpu.sync_copy(src_hbm.at[pl.ds(off, CH)], idx_v)
            pltpu.sync_copy(dst_hbm.at[pl.ds(off, CH)], didx_v)
            pltpu.async_copy(s_hbm.at[idx_v], rows_v, sem).wait()
            pltpu.sync_copy(rows_v, shared.at[didx_v], add=True)
            return carry

        lax.fori_loop(0, NCH, step, 0)
        plsc.subcore_barrier()

        for k in range(NZ):
            row = sid * RPT + k * ZCH
            pltpu.sync_copy(shared.at[pl.ds(row, ZCH)], zv)
            pltpu.sync_copy(zv, out_hbm.at[pl.ds(cid * NP + row, ZCH)])

    return agg


@functools.partial(
    pl.kernel,
    out_type=jax.ShapeDtypeStruct((NC * NP, WD), jnp.float32),
    mesh=_sc_mesh(),
    scratch_types=[
        pltpu.VMEM_SHARED((NP, WD), jnp.float32),
        pltpu.VMEM((CH,), jnp.int32),
        pltpu.VMEM((CH, WD), jnp.float32),
        pltpu.VMEM((ZCH, WD), jnp.float32),
    ],
)
def _deg_kernel(dst_hbm, ones_hbm, zeros_hbm, out_hbm,
                shared, didx_v, ones_v, zv):
    cid = lax.axis_index("c")
    sid = lax.axis_index("s")
    wid = sid * NC + cid

    pltpu.sync_copy(zeros_hbm, zv)
    pltpu.sync_copy(ones_hbm, ones_v)
    for k in range(NZ):
        pltpu.sync_copy(zv, shared.at[pl.ds(sid * RPT + k * ZCH, ZCH)])
    plsc.subcore_barrier()

    base = wid * EPW

    def step(j, carry):
        off = pl.multiple_of(base + j * CH, 8)
        pltpu.sync_copy(dst_hbm.at[pl.ds(off, CH)], didx_v)
        pltpu.sync_copy(ones_v, shared.at[didx_v], add=True)
        return carry

    lax.fori_loop(0, NCH, step, 0)
    plsc.subcore_barrier()

    for k in range(NZ):
        row = sid * RPT + k * ZCH
        pltpu.sync_copy(shared.at[pl.ds(row, ZCH)], zv)
        pltpu.sync_copy(zv, out_hbm.at[pl.ds(cid * NP + row, ZCH)])


def _mm1_body(x_ref, w_ref, d0_ref, d1_ref, s_ref, dinv_ref):
    deg = d0_ref[:, :1] + d1_ref[:, :1]
    dinv = lax.rsqrt(deg + 1.0)
    h = jnp.dot(x_ref[...], w_ref[...], preferred_element_type=jnp.float32)
    s_ref[...] = h * dinv
    dinv_ref[...] = jnp.broadcast_to(dinv, dinv_ref.shape)


def _mm1(xp, W1, deg0, deg1):
    return pl.pallas_call(
        _mm1_body,
        grid=(GR,),
        in_specs=[
            pl.BlockSpec((BR, D_IN), lambda i: (i, 0)),
            pl.BlockSpec((D_IN, D_H), lambda i: (0, 0)),
            pl.BlockSpec((BR, WD), lambda i: (i, 0)),
            pl.BlockSpec((BR, WD), lambda i: (i, 0)),
        ],
        out_specs=[
            pl.BlockSpec((BR, D_H), lambda i: (i, 0)),
            pl.BlockSpec((BR, 8), lambda i: (i, 0)),
        ],
        out_shape=[
            jax.ShapeDtypeStruct((NP, D_H), jnp.float32),
            jax.ShapeDtypeStruct((NP, 8), jnp.float32),
        ],
    )(xp, W1, deg0, deg1)


def _mid_body(a0_ref, a1_ref, s_ref, dinv_ref, b_ref, w_ref, o_ref):
    dinv = dinv_ref[:, :1]
    t = (a0_ref[...] + a1_ref[...] + s_ref[...]) * dinv + b_ref[...]
    t = jnp.maximum(t, 0.0)
    o_ref[...] = jnp.dot(t, w_ref[...],
                         preferred_element_type=jnp.float32) * dinv


def _mm_mid(a0, a1, s, dinv, b, W):
    d_in = s.shape[1]
    d_out = W.shape[1]
    return pl.pallas_call(
        _mid_body,
        grid=(GR,),
        in_specs=[
            pl.BlockSpec((BR, d_in), lambda i: (i, 0)),
            pl.BlockSpec((BR, d_in), lambda i: (i, 0)),
            pl.BlockSpec((BR, d_in), lambda i: (i, 0)),
            pl.BlockSpec((BR, 8), lambda i: (i, 0)),
            pl.BlockSpec((1, d_in), lambda i: (0, 0)),
            pl.BlockSpec((d_in, d_out), lambda i: (0, 0)),
        ],
        out_specs=pl.BlockSpec((BR, d_out), lambda i: (i, 0)),
        out_shape=jax.ShapeDtypeStruct((NP, d_out), jnp.float32),
    )(a0, a1, s, dinv, b, W)


def _fin_body(a0_ref, a1_ref, s_ref, dinv_ref, b_ref, o_ref):
    dinv = dinv_ref[:, :1]
    t = (a0_ref[...] + a1_ref[...] + s_ref[...]) * dinv + b_ref[...]
    col = lax.broadcasted_iota(jnp.int32, t.shape, 1)
    valid = col < D_OUT
    tm = jnp.where(valid, t, -jnp.inf)
    m = jnp.max(tm, axis=1, keepdims=True)
    ex = jnp.where(valid, jnp.exp(t - m), 0.0)
    lse = jnp.log(jnp.sum(ex, axis=1, keepdims=True))
    o_ref[...] = t - m - lse


def _fin(a0, a1, s, dinv, b):
    return pl.pallas_call(
        _fin_body,
        grid=(GR,),
        in_specs=[
            pl.BlockSpec((BR, DP), lambda i: (i, 0)),
            pl.BlockSpec((BR, DP), lambda i: (i, 0)),
            pl.BlockSpec((BR, DP), lambda i: (i, 0)),
            pl.BlockSpec((BR, 8), lambda i: (i, 0)),
            pl.BlockSpec((1, DP), lambda i: (0, 0)),
        ],
        out_specs=pl.BlockSpec((BR, DP), lambda i: (i, 0)),
        out_shape=jax.ShapeDtypeStruct((NP, DP), jnp.float32),
    )(a0, a1, s, dinv, b)


_agg_h = _make_agg(D_H)
_agg_p = _make_agg(DP)


@jax.jit
def kernel(x, edge_index, W1, b1, W2, b2, Wo, bo):
    src = edge_index[0].astype(jnp.int32)
    dst = edge_index[1].astype(jnp.int32)
    xp = jnp.pad(x, ((0, NP - N), (0, 0)))
    Wop = jnp.pad(Wo, ((0, 0), (0, DP - D_OUT)))
    b1r = b1.reshape(1, D_H)
    b2r = b2.reshape(1, D_H)
    bor = jnp.pad(bo, (0, DP - D_OUT)).reshape(1, DP)

    ones_w = jnp.ones((CH, WD), jnp.float32)
    zeros_w = jnp.zeros((ZCH, WD), jnp.float32)
    zeros_h = jnp.zeros((ZCH, D_H), jnp.float32)
    zeros_p = jnp.zeros((ZCH, DP), jnp.float32)

    degp = _deg_kernel(dst, ones_w, zeros_w)
    deg0, deg1 = degp[:NP], degp[NP:]

    s1, dinv = _mm1(xp, W1, deg0, deg1)
    a1 = _agg_h(s1, src, dst, zeros_h)
    s2 = _mm_mid(a1[:NP], a1[NP:], s1, dinv, b1r, W2)
    a2 = _agg_h(s2, src, dst, zeros_h)
    s3 = _mm_mid(a2[:NP], a2[NP:], s2, dinv, b2r, Wop)
    a3 = _agg_p(s3, src, dst, zeros_p)
    outp = _fin(a3[:NP], a3[NP:], s3, dinv, bor)
    return outp[:N, :D_OUT]


# speedup vs baseline: 19.0685x; 1.8543x over previous
import functools

import jax
import jax.numpy as jnp
from jax import lax
from jax.experimental import pallas as pl
from jax.experimental.pallas import tpu as pltpu
from jax.experimental.pallas import tpu_sc as plsc

N = 10000
E = 320000
D_IN = 128
D_H = 128
D_OUT = 40
DP = 128
NP = 10240

NC = 2
NS = 16
NW = NC * NS
EPW = E // NW
CHF = 128
NCHF = EPW // CHF
CHT = EPW - NCHF * CHF
NB = 2
RPT = NP // NS
ZCH = 80
NZ = RPT // ZCH

WD = 128

BR = 1024
GR = NP // BR


def _sc_mesh():
    return plsc.VectorSubcoreMesh(core_axis_name="c", subcore_axis_name="s")


def _zero_shared(shared, zeros_hbm, zv, sid, sem):
    pltpu.sync_copy(zeros_hbm, zv)
    for k in range(NZ):
        pltpu.async_copy(zv, shared.at[pl.ds(sid * RPT + k * ZCH, ZCH)], sem)
    for k in range(NZ):
        pltpu.make_async_copy(
            zv, shared.at[pl.ds(sid * RPT + k * ZCH, ZCH)], sem).wait()


def _drain_shared(shared, out_hbm, st0, st1, cid, sid, sem0, sem1):
    stages = (st0, st1)
    sems = (sem0, sem1)
    pltpu.async_copy(shared.at[pl.ds(sid * RPT, ZCH)], stages[0], sems[0])
    for k in range(NZ):
        b = k % 2
        row = sid * RPT + k * ZCH
        if k + 1 < NZ:
            nrow = row + ZCH
            pltpu.async_copy(shared.at[pl.ds(nrow, ZCH)],
                             stages[(k + 1) % 2], sems[(k + 1) % 2])
        pltpu.make_async_copy(shared.at[pl.ds(row, ZCH)],
                              stages[b], sems[b]).wait()
        pltpu.sync_copy(stages[b], out_hbm.at[pl.ds(cid * NP + row, ZCH)])


def _make_agg(D):
    @functools.partial(
        pl.kernel,
        out_type=jax.ShapeDtypeStruct((NC * NP, D), jnp.float32),
        mesh=_sc_mesh(),
        scratch_types=[
            pltpu.VMEM_SHARED((NP, D), jnp.float32),
            pltpu.VMEM((CHF,), jnp.int32),
            pltpu.VMEM((CHF,), jnp.int32),
            pltpu.VMEM((CHF,), jnp.int32),
            pltpu.VMEM((CHF,), jnp.int32),
            pltpu.VMEM((CHF, D), jnp.float32),
            pltpu.VMEM((CHF, D), jnp.float32),
            pltpu.VMEM((CHT,), jnp.int32),
            pltpu.VMEM((CHT,), jnp.int32),
            pltpu.VMEM((CHT, D), jnp.float32),
            pltpu.VMEM((ZCH, D), jnp.float32),
            pltpu.SemaphoreType.DMA,
            pltpu.SemaphoreType.DMA,
        ],
    )
    def agg(s_hbm, src_hbm, dst_hbm, zeros_hbm, out_hbm,
            shared, si0, si1, di0, di1, rw0, rw1, sit, dit, rwt,
            zv, sem0, sem1):
        cid = lax.axis_index("c")
        sid = lax.axis_index("s")
        wid = sid * NC + cid
        sidx = (si0, si1)
        didx = (di0, di1)
        rows = (rw0, rw1)
        sems = (sem0, sem1)

        _zero_shared(shared, zeros_hbm, zv, sid, sem0)
        plsc.subcore_barrier()

        base = wid * EPW

        for b in range(NB):
            off = pl.multiple_of(base + b * CHF, 8)
            pltpu.sync_copy(src_hbm.at[pl.ds(off, CHF)], sidx[b])
            pltpu.sync_copy(dst_hbm.at[pl.ds(off, CHF)], didx[b])
            pltpu.async_copy(s_hbm.at[sidx[b]], rows[b], sems[b])

        @pl.loop(0, NCHF - NB, step=NB)
        def _(g):
            for b in range(NB):
                pltpu.make_async_copy(
                    s_hbm.at[sidx[b]], rows[b], sems[b]).wait()
                pltpu.sync_copy(rows[b], shared.at[didx[b]], add=True)
                off = pl.multiple_of(base + (g + b + NB) * CHF, 8)
                pltpu.sync_copy(src_hbm.at[pl.ds(off, CHF)], sidx[b])
                pltpu.sync_copy(dst_hbm.at[pl.ds(off, CHF)], didx[b])
                pltpu.async_copy(s_hbm.at[sidx[b]], rows[b], sems[b])

        for b in range(NB):
            pltpu.make_async_copy(
                s_hbm.at[sidx[b]], rows[b], sems[b]).wait()
            pltpu.sync_copy(rows[b], shared.at[didx[b]], add=True)

        offt = pl.multiple_of(base + NCHF * CHF, 8)
        pltpu.sync_copy(src_hbm.at[pl.ds(offt, CHT)], sit)
        pltpu.sync_copy(dst_hbm.at[pl.ds(offt, CHT)], dit)
        pltpu.async_copy(s_hbm.at[sit], rwt, sem0).wait()
        pltpu.sync_copy(rwt, shared.at[dit], add=True)

        plsc.subcore_barrier()
        _drain_shared(shared, out_hbm, rw0.at[pl.ds(0, ZCH)],
                      rw1.at[pl.ds(0, ZCH)], cid, sid, sem0, sem1)

    return agg


@functools.partial(
    pl.kernel,
    out_type=jax.ShapeDtypeStruct((NC * NP, WD), jnp.float32),
    mesh=_sc_mesh(),
    scratch_types=[
        pltpu.VMEM_SHARED((NP, WD), jnp.float32),
        pltpu.VMEM((CHF,), jnp.int32),
        pltpu.VMEM((CHF,), jnp.int32),
        pltpu.VMEM((CHT,), jnp.int32),
        pltpu.VMEM((CHF, WD), jnp.float32),
        pltpu.VMEM((ZCH, WD), jnp.float32),
        pltpu.VMEM((ZCH, WD), jnp.float32),
        pltpu.SemaphoreType.DMA,
        pltpu.SemaphoreType.DMA,
    ],
)
def _deg_kernel(dst_hbm, ones_hbm, zeros_hbm, out_hbm,
                shared, di0, di1, dit, ones_v, zv, zv1, sem0, sem1):
    cid = lax.axis_index("c")
    sid = lax.axis_index("s")
    wid = sid * NC + cid
    didx = (di0, di1)
    sems = (sem0, sem1)

    pltpu.sync_copy(ones_hbm, ones_v)
    _zero_shared(shared, zeros_hbm, zv, sid, sem0)
    plsc.subcore_barrier()

    base = wid * EPW

    for b in range(NB):
        off = pl.multiple_of(base + b * CHF, 8)
        pltpu.async_copy(dst_hbm.at[pl.ds(off, CHF)], didx[b], sems[b])

    @pl.loop(0, NCHF - NB, step=NB)
    def _(g):
        for b in range(NB):
            off0 = pl.multiple_of(base + (g + b) * CHF, 8)
            pltpu.make_async_copy(
                dst_hbm.at[pl.ds(off0, CHF)], didx[b], sems[b]).wait()
            pltpu.sync_copy(ones_v, shared.at[didx[b]], add=True)
            off = pl.multiple_of(base + (g + b + NB) * CHF, 8)
            pltpu.async_copy(dst_hbm.at[pl.ds(off, CHF)], didx[b], sems[b])

    for b in range(NB):
        off0 = pl.multiple_of(base + (NCHF - NB + b) * CHF, 8)
        pltpu.make_async_copy(
            dst_hbm.at[pl.ds(off0, CHF)], didx[b], sems[b]).wait()
        pltpu.sync_copy(ones_v, shared.at[didx[b]], add=True)

    offt = pl.multiple_of(base + NCHF * CHF, 8)
    pltpu.sync_copy(dst_hbm.at[pl.ds(offt, CHT)], dit)
    pltpu.sync_copy(ones_v.at[pl.ds(0, CHT)], shared.at[dit], add=True)

    plsc.subcore_barrier()
    _drain_shared(shared, out_hbm, zv, zv1, cid, sid, sem0, sem1)


def _mm1_body(x_ref, w_ref, d0_ref, d1_ref, s_ref, dinv_ref):
    deg = d0_ref[:, :1] + d1_ref[:, :1]
    dinv = lax.rsqrt(deg + 1.0)
    h = jnp.dot(x_ref[...], w_ref[...], preferred_element_type=jnp.float32)
    s_ref[...] = h * dinv
    dinv_ref[...] = jnp.broadcast_to(dinv, dinv_ref.shape)


def _mm1(xp, W1, deg0, deg1):
    return pl.pallas_call(
        _mm1_body,
        grid=(GR,),
        in_specs=[
            pl.BlockSpec((BR, D_IN), lambda i: (i, 0)),
            pl.BlockSpec((D_IN, D_H), lambda i: (0, 0)),
            pl.BlockSpec((BR, WD), lambda i: (i, 0)),
            pl.BlockSpec((BR, WD), lambda i: (i, 0)),
        ],
        out_specs=[
            pl.BlockSpec((BR, D_H), lambda i: (i, 0)),
            pl.BlockSpec((BR, 8), lambda i: (i, 0)),
        ],
        out_shape=[
            jax.ShapeDtypeStruct((NP, D_H), jnp.float32),
            jax.ShapeDtypeStruct((NP, 8), jnp.float32),
        ],
    )(xp, W1, deg0, deg1)


def _mid_body(a0_ref, a1_ref, s_ref, dinv_ref, b_ref, w_ref, o_ref):
    dinv = dinv_ref[:, :1]
    t = (a0_ref[...] + a1_ref[...] + s_ref[...]) * dinv + b_ref[...]
    t = jnp.maximum(t, 0.0)
    o_ref[...] = jnp.dot(t, w_ref[...],
                         preferred_element_type=jnp.float32) * dinv


def _mm_mid(a0, a1, s, dinv, b, W):
    d_in = s.shape[1]
    d_out = W.shape[1]
    return pl.pallas_call(
        _mid_body,
        grid=(GR,),
        in_specs=[
            pl.BlockSpec((BR, d_in), lambda i: (i, 0)),
            pl.BlockSpec((BR, d_in), lambda i: (i, 0)),
            pl.BlockSpec((BR, d_in), lambda i: (i, 0)),
            pl.BlockSpec((BR, 8), lambda i: (i, 0)),
            pl.BlockSpec((1, d_in), lambda i: (0, 0)),
            pl.BlockSpec((d_in, d_out), lambda i: (0, 0)),
        ],
        out_specs=pl.BlockSpec((BR, d_out), lambda i: (i, 0)),
        out_shape=jax.ShapeDtypeStruct((NP, d_out), jnp.float32),
    )(a0, a1, s, dinv, b, W)


def _fin_body(a0_ref, a1_ref, s_ref, dinv_ref, b_ref, o_ref):
    dinv = dinv_ref[:, :1]
    t = (a0_ref[...] + a1_ref[...] + s_ref[...]) * dinv + b_ref[...]
    col = lax.broadcasted_iota(jnp.int32, t.shape, 1)
    valid = col < D_OUT
    tm = jnp.where(valid, t, -jnp.inf)
    m = jnp.max(tm, axis=1, keepdims=True)
    ex = jnp.where(valid, jnp.exp(t - m), 0.0)
    lse = jnp.log(jnp.sum(ex, axis=1, keepdims=True))
    o_ref[...] = t - m - lse


def _fin(a0, a1, s, dinv, b):
    return pl.pallas_call(
        _fin_body,
        grid=(GR,),
        in_specs=[
            pl.BlockSpec((BR, DP), lambda i: (i, 0)),
            pl.BlockSpec((BR, DP), lambda i: (i, 0)),
            pl.BlockSpec((BR, DP), lambda i: (i, 0)),
            pl.BlockSpec((BR, 8), lambda i: (i, 0)),
            pl.BlockSpec((1, DP), lambda i: (0, 0)),
        ],
        out_specs=pl.BlockSpec((BR, DP), lambda i: (i, 0)),
        out_shape=jax.ShapeDtypeStruct((NP, DP), jnp.float32),
    )(a0, a1, s, dinv, b)


_agg_h = _make_agg(D_H)


@jax.jit
def kernel(x, edge_index, W1, b1, W2, b2, Wo, bo):
    src_i = edge_index[0].astype(jnp.int32)
    dst_i = edge_index[1].astype(jnp.int32)
    xp = jnp.pad(x, ((0, NP - N), (0, 0)))
    Wop = jnp.pad(Wo, ((0, 0), (0, DP - D_OUT)))
    b1r = b1.reshape(1, D_H)
    b2r = b2.reshape(1, D_H)
    bor = jnp.pad(bo, (0, DP - D_OUT)).reshape(1, DP)

    ones_w = jnp.ones((CHF, WD), jnp.float32)
    zeros_w = jnp.zeros((ZCH, WD), jnp.float32)
    zeros_h = jnp.zeros((ZCH, D_H), jnp.float32)

    degp = _deg_kernel(dst_i, ones_w, zeros_w)
    deg0, deg1 = degp[:NP], degp[NP:]

    s1, dinv = _mm1(xp, W1, deg0, deg1)
    a1 = _agg_h(s1, src_i, dst_i, zeros_h)
    s2 = _mm_mid(a1[:NP], a1[NP:], s1, dinv, b1r, W2)
    a2 = _agg_h(s2, src_i, dst_i, zeros_h)
    s3 = _mm_mid(a2[:NP], a2[NP:], s2, dinv, b2r, Wop)
    a3 = _agg_h(s3, src_i, dst_i, zeros_h)
    outp = _fin(a3[:NP], a3[NP:], s3, dinv, bor)
    return outp[:N, :D_OUT]
